# Initial kernel scaffold; baseline (speedup 1.0000x reference)
import jax
import jax.numpy as jnp
from jax.experimental import pallas as pl


def kernel(x, indices):
    raise NotImplementedError("write your pallas kernel here")


# speedup vs baseline: 2.4179x; 2.4179x over previous
import jax
import jax.numpy as jnp
from jax import lax
from jax.experimental import pallas as pl
from jax.experimental.pallas import tpu as pltpu
from jax.experimental.pallas import tpu_sc as plsc

N_EDGES = 160000
D_FEAT = 256
DIM_SIZE = 10000

NC = 2
NS = 16
L = 16
DH = D_FEAT // NC
EPW = N_EDGES // NS
B = 80
NB = EPW // B

SEG_PER_SC = 5120
GARBAGE = SEG_PER_SC
CNT_ROWS = SEG_PER_SC + 128
ZPW1 = CNT_ROWS // NS
CPW = SEG_PER_SC // NS

NSEG_PAD = 10240
ZPW2 = NSEG_PAD // NS
ZB = 64
DB = 16
NDB = DIM_SIZE // DB


def _counts_body(idx_hbm, cnt_hbm, cnts_sh, idx_v, idx2_v, ones_v, zbuf):
    c = lax.axis_index("c")
    s = lax.axis_index("s")
    lo = c * SEG_PER_SC

    zero_v = jnp.zeros((L,), jnp.float32)
    one_v = jnp.ones((L,), jnp.float32)

    def _fill_ones(i, _):
        def _col(k, _):
            ones_v[i, pl.ds(k * L, L)] = one_v
            return 0
        lax.fori_loop(0, DH // L, _col, 0)
        return 0
    lax.fori_loop(0, B, _fill_ones, 0)

    def _fill(i, _):
        def _col(k, _):
            zbuf[i, pl.ds(k * L, L)] = zero_v
            return 0
        lax.fori_loop(0, DH // L, _col, 0)
        return 0
    lax.fori_loop(0, ZB, _fill, 0)

    z0 = s * ZPW1
    for t in range(ZPW1 // ZB):
        pltpu.sync_copy(zbuf, cnts_sh.at[pl.ds(z0 + t * ZB, ZB)])
    pltpu.sync_copy(zbuf.at[pl.ds(0, ZPW1 % ZB)],
                    cnts_sh.at[pl.ds(z0 + (ZPW1 // ZB) * ZB, ZPW1 % ZB)])
    plsc.subcore_barrier()

    def _batch(j, _):
        base = s * EPW + j * B
        pltpu.sync_copy(idx_hbm.at[pl.ds(base, B)], idx_v)

        for k in range(B // L):
            v = idx_v[pl.ds(k * L, L)]
            t = v - lo
            valid = jnp.logical_and(t >= 0, t < SEG_PER_SC)
            idx2_v[pl.ds(k * L, L)] = jnp.where(valid, t, GARBAGE)

        pltpu.sync_copy(ones_v, cnts_sh.at[idx2_v], add=True)
        return 0
    lax.fori_loop(0, NB, _batch, 0)
    plsc.subcore_barrier()

    g0 = c * SEG_PER_SC + s * CPW
    for t in range(CPW // ZB):
        pltpu.sync_copy(cnts_sh.at[pl.ds(s * CPW + t * ZB, ZB)],
                        zbuf)
        pltpu.sync_copy(zbuf, cnt_hbm.at[pl.ds(g0 + t * ZB, ZB)])


def _sums_body(x_hbm, idx_hbm, cnt_hbm, out_hbm, sums_sh, idx_v, xbuf,
               zbuf, rowbuf, cbuf):
    c = lax.axis_index("c")
    s = lax.axis_index("s")

    zero_v = jnp.zeros((L,), jnp.float32)

    def _fill(i, _):
        def _col(k, _):
            zbuf[i, pl.ds(k * L, L)] = zero_v
            return 0
        lax.fori_loop(0, DH // L, _col, 0)
        return 0
    lax.fori_loop(0, ZB, _fill, 0)

    z0 = s * ZPW2
    for t in range(ZPW2 // ZB):
        pltpu.sync_copy(zbuf, sums_sh.at[pl.ds(z0 + t * ZB, ZB)])
    plsc.subcore_barrier()

    def _batch(j, _):
        base = s * EPW + j * B
        pltpu.sync_copy(idx_hbm.at[pl.ds(base, B)], idx_v)
        pltpu.sync_copy(x_hbm.at[pl.ds(base, B), pl.ds(c * DH, DH)], xbuf)
        pltpu.sync_copy(xbuf, sums_sh.at[idx_v], add=True)
        return 0
    lax.fori_loop(0, NB, _batch, 0)
    plsc.subcore_barrier()

    nb_mine = jnp.where(s == 0, NDB - (NS - 1) * (NDB // NS), NDB // NS)

    def _div_batch(m, _):
        r0 = (s + m * NS) * DB
        pltpu.sync_copy(sums_sh.at[pl.ds(r0, DB)], rowbuf)
        pltpu.sync_copy(cnt_hbm.at[pl.ds(r0, DB)], cbuf)

        def _row(i, _):
            inv = 1.0 / jnp.maximum(cbuf[i, pl.ds(0, L)], 1.0)

            def _col(k, _):
                rowbuf[i, pl.ds(k * L, L)] = rowbuf[i, pl.ds(k * L, L)] * inv
                return 0
            lax.fori_loop(0, DH // L, _col, 0)
            return 0
        lax.fori_loop(0, DB, _row, 0)

        pltpu.sync_copy(rowbuf,
                        out_hbm.at[pl.ds(r0, DB), pl.ds(c * DH, DH)])
        return 0
    lax.fori_loop(0, nb_mine, _div_batch, 0)


def kernel(x, indices):
    idx = indices.astype(jnp.int32)
    mesh = plsc.VectorSubcoreMesh(
        core_axis_name="c", subcore_axis_name="s",
        num_cores=NC, num_subcores=NS)

    k1 = pl.kernel(
        _counts_body,
        out_type=jax.ShapeDtypeStruct((2 * SEG_PER_SC, DH), jnp.float32),
        mesh=mesh,
        scratch_types=[
            pltpu.VMEM_SHARED((CNT_ROWS, DH), jnp.float32),
            pltpu.VMEM((B,), jnp.int32),
            pltpu.VMEM((B,), jnp.int32),
            pltpu.VMEM((B, DH), jnp.float32),
            pltpu.VMEM((ZB, DH), jnp.float32),
        ],
    )
    counts = k1(idx)

    k2 = pl.kernel(
        _sums_body,
        out_type=jax.ShapeDtypeStruct((DIM_SIZE, D_FEAT), jnp.float32),
        mesh=mesh,
        scratch_types=[
            pltpu.VMEM_SHARED((NSEG_PAD, DH), jnp.float32),
            pltpu.VMEM((B,), jnp.int32),
            pltpu.VMEM((B, DH), jnp.float32),
            pltpu.VMEM((ZB, DH), jnp.float32),
            pltpu.VMEM((DB, DH), jnp.float32),
            pltpu.VMEM((DB, DH), jnp.float32),
        ],
    )
    return k2(x, idx, counts)


# speedup vs baseline: 3.1874x; 1.3182x over previous
import jax
import jax.numpy as jnp
from jax import lax
from jax.experimental import pallas as pl
from jax.experimental.pallas import tpu as pltpu
from jax.experimental.pallas import tpu_sc as plsc

N_EDGES = 160000
D_FEAT = 256
DIM_SIZE = 10000

NC = 2
NS = 16
L = 16
DH = D_FEAT // NC
EPW = N_EDGES // NS
B = 80
NB = EPW // B

SEG_PER_SC = 5120
GARBAGE = SEG_PER_SC
CNT_ROWS = SEG_PER_SC + 128
ZPW1 = CNT_ROWS // NS
CPW = SEG_PER_SC // NS

NSEG_PAD = 10240
ZPW2 = NSEG_PAD // NS
ZB = 64
DB = 16
NDB = DIM_SIZE // DB


def _counts_body(idx_hbm, cnt_hbm, cnts_sh, idx_v, idx2_v, ones_v, zbuf):
    c = lax.axis_index("c")
    s = lax.axis_index("s")
    lo = c * SEG_PER_SC

    zero_v = jnp.zeros((L,), jnp.float32)
    one_v = jnp.ones((L,), jnp.float32)

    def _fill_ones(i, _):
        def _col(k, _):
            ones_v[i, pl.ds(k * L, L)] = one_v
            return 0
        lax.fori_loop(0, DH // L, _col, 0)
        return 0
    lax.fori_loop(0, B, _fill_ones, 0)

    def _fill(i, _):
        def _col(k, _):
            zbuf[i, pl.ds(k * L, L)] = zero_v
            return 0
        lax.fori_loop(0, DH // L, _col, 0)
        return 0
    lax.fori_loop(0, ZB, _fill, 0)

    z0 = s * ZPW1
    for t in range(ZPW1 // ZB):
        pltpu.sync_copy(zbuf, cnts_sh.at[pl.ds(z0 + t * ZB, ZB)])
    pltpu.sync_copy(zbuf.at[pl.ds(0, ZPW1 % ZB)],
                    cnts_sh.at[pl.ds(z0 + (ZPW1 // ZB) * ZB, ZPW1 % ZB)])
    plsc.subcore_barrier()

    def _batch(j, _):
        base = s * EPW + j * B
        pltpu.sync_copy(idx_hbm.at[pl.ds(base, B)], idx_v)

        for k in range(B // L):
            v = idx_v[pl.ds(k * L, L)]
            t = v - lo
            valid = jnp.logical_and(t >= 0, t < SEG_PER_SC)
            idx2_v[pl.ds(k * L, L)] = jnp.where(valid, t, GARBAGE)

        pltpu.sync_copy(ones_v, cnts_sh.at[idx2_v], add=True)
        return 0
    lax.fori_loop(0, NB, _batch, 0)
    plsc.subcore_barrier()

    g0 = c * SEG_PER_SC + s * CPW
    for t in range(CPW // ZB):
        pltpu.sync_copy(cnts_sh.at[pl.ds(s * CPW + t * ZB, ZB)],
                        zbuf)
        pltpu.sync_copy(zbuf, cnt_hbm.at[pl.ds(g0 + t * ZB, ZB)])


def _sums_body(x_hbm, idx_hbm, cnt_hbm, out_hbm, sums_sh, idx_v, idx_b,
               xbuf, xbuf_b, zbuf, rowbuf, cbuf, sem_a, sem_b):
    c = lax.axis_index("c")
    s = lax.axis_index("s")

    zero_v = jnp.zeros((L,), jnp.float32)

    def _fill(i, _):
        def _col(k, _):
            zbuf[i, pl.ds(k * L, L)] = zero_v
            return 0
        lax.fori_loop(0, DH // L, _col, 0)
        return 0
    lax.fori_loop(0, ZB, _fill, 0)

    z0 = s * ZPW2
    for t in range(ZPW2 // ZB):
        pltpu.sync_copy(zbuf, sums_sh.at[pl.ds(z0 + t * ZB, ZB)])
    plsc.subcore_barrier()

    e0 = s * EPW
    col = c * DH

    def _loads(j, ibuf, xb, sem):
        pltpu.async_copy(idx_hbm.at[pl.ds(e0 + j * B, B)], ibuf, sem)
        pltpu.async_copy(x_hbm.at[pl.ds(e0 + j * B, B), pl.ds(col, DH)],
                         xb, sem)

    def _waits(j, ibuf, xb, sem):
        pltpu.make_async_copy(idx_hbm.at[pl.ds(e0 + j * B, B)], ibuf,
                              sem).wait()
        pltpu.make_async_copy(x_hbm.at[pl.ds(e0 + j * B, B),
                                       pl.ds(col, DH)], xb, sem).wait()

    _loads(0, idx_v, xbuf, sem_a)

    def _pair(j2, _):
        j = j2 * 2
        _waits(j, idx_v, xbuf, sem_a)
        _loads(j + 1, idx_b, xbuf_b, sem_b)
        pltpu.sync_copy(xbuf, sums_sh.at[idx_v], add=True)
        _waits(j + 1, idx_b, xbuf_b, sem_b)
        _loads(j + 2, idx_v, xbuf, sem_a)
        pltpu.sync_copy(xbuf_b, sums_sh.at[idx_b], add=True)
        return 0
    lax.fori_loop(0, NB // 2, _pair, 0)

    _waits(NB - 1, idx_v, xbuf, sem_a)
    pltpu.sync_copy(xbuf, sums_sh.at[idx_v], add=True)
    plsc.subcore_barrier()

    nb_mine = jnp.where(s == 0, NDB - (NS - 1) * (NDB // NS), NDB // NS)

    def _div_batch(m, _):
        r0 = (s + m * NS) * DB
        pltpu.sync_copy(sums_sh.at[pl.ds(r0, DB)], rowbuf)
        pltpu.sync_copy(cnt_hbm.at[pl.ds(r0, DB)], cbuf)

        def _row(i, _):
            inv = 1.0 / jnp.maximum(cbuf[i, pl.ds(0, L)], 1.0)

            def _col(k, _):
                rowbuf[i, pl.ds(k * L, L)] = rowbuf[i, pl.ds(k * L, L)] * inv
                return 0
            lax.fori_loop(0, DH // L, _col, 0)
            return 0
        lax.fori_loop(0, DB, _row, 0)

        pltpu.sync_copy(rowbuf,
                        out_hbm.at[pl.ds(r0, DB), pl.ds(c * DH, DH)])
        return 0
    lax.fori_loop(0, nb_mine, _div_batch, 0)


def kernel(x, indices):
    idx = indices.astype(jnp.int32)
    mesh = plsc.VectorSubcoreMesh(
        core_axis_name="c", subcore_axis_name="s",
        num_cores=NC, num_subcores=NS)

    k1 = pl.kernel(
        _counts_body,
        out_type=jax.ShapeDtypeStruct((2 * SEG_PER_SC, DH), jnp.float32),
        mesh=mesh,
        scratch_types=[
            pltpu.VMEM_SHARED((CNT_ROWS, DH), jnp.float32),
            pltpu.VMEM((B,), jnp.int32),
            pltpu.VMEM((B,), jnp.int32),
            pltpu.VMEM((B, DH), jnp.float32),
            pltpu.VMEM((ZB, DH), jnp.float32),
        ],
    )
    counts = k1(idx)

    k2 = pl.kernel(
        _sums_body,
        out_type=jax.ShapeDtypeStruct((DIM_SIZE, D_FEAT), jnp.float32),
        mesh=mesh,
        scratch_types=[
            pltpu.VMEM_SHARED((NSEG_PAD, DH), jnp.float32),
            pltpu.VMEM((B,), jnp.int32),
            pltpu.VMEM((B,), jnp.int32),
            pltpu.VMEM((B, DH), jnp.float32),
            pltpu.VMEM((B, DH), jnp.float32),
            pltpu.VMEM((ZB, DH), jnp.float32),
            pltpu.VMEM((DB, DH), jnp.float32),
            pltpu.VMEM((DB, DH), jnp.float32),
            pltpu.SemaphoreType.DMA,
            pltpu.SemaphoreType.DMA,
        ],
    )
    return k2(x, idx, counts)


# speedup vs baseline: 3.7920x; 1.1897x over previous
import jax
import jax.numpy as jnp
from jax import lax
from jax.experimental import pallas as pl
from jax.experimental.pallas import tpu as pltpu
from jax.experimental.pallas import tpu_sc as plsc

N_EDGES = 160000
D_FEAT = 256
DIM_SIZE = 10000

NC = 2
NS = 16
L = 16
DH = D_FEAT // NC
EPW = N_EDGES // NS
B = 80
NB = EPW // B

SEG_PER_SC = 5120
GARBAGE = SEG_PER_SC
CNT_ROWS = SEG_PER_SC + 128
ZPW1 = CNT_ROWS // NS
CPW = SEG_PER_SC // NS

NSEG_PAD = 10240
ZPW2 = NSEG_PAD // NS
ZB = 64
DB = 16
NDB = DIM_SIZE // DB


def _counts_body(idx_hbm, cnt_hbm, cnts_sh, idx_v, idx_b, idx2_v, ones_v,
                 zbuf, sem_a, sem_b):
    c = lax.axis_index("c")
    s = lax.axis_index("s")
    lo = c * SEG_PER_SC

    zero_v = jnp.zeros((L,), jnp.float32)
    one_v = jnp.ones((L,), jnp.float32)

    def _fill_ones(i, _):
        def _col(k, _):
            ones_v[i, pl.ds(k * L, L)] = one_v
            return 0
        lax.fori_loop(0, DH // L, _col, 0)
        return 0
    lax.fori_loop(0, B, _fill_ones, 0)

    def _fill(i, _):
        def _col(k, _):
            zbuf[i, pl.ds(k * L, L)] = zero_v
            return 0
        lax.fori_loop(0, DH // L, _col, 0)
        return 0
    lax.fori_loop(0, ZB, _fill, 0)

    z0 = s * ZPW1
    for t in range(ZPW1 // ZB):
        pltpu.sync_copy(zbuf, cnts_sh.at[pl.ds(z0 + t * ZB, ZB)])
    pltpu.sync_copy(zbuf.at[pl.ds(0, ZPW1 % ZB)],
                    cnts_sh.at[pl.ds(z0 + (ZPW1 // ZB) * ZB, ZPW1 % ZB)])
    plsc.subcore_barrier()

    e0 = s * EPW
    hi = lo + SEG_PER_SC

    def _loads(j, buf, sem):
        pltpu.async_copy(idx_hbm.at[pl.ds(e0 + j * B, B)], buf, sem)

    def _scatter(j, buf, sem):
        pltpu.make_async_copy(idx_hbm.at[pl.ds(e0 + j * B, B)], buf,
                              sem).wait()
        first = buf[pl.ds(0, L)][0]
        last = buf[pl.ds(B - L, L)][L - 1]

        @pl.when(jnp.logical_and(first < hi, last >= lo))
        def _():
            for k in range(B // L):
                v = buf[pl.ds(k * L, L)]
                t = v - lo
                valid = jnp.logical_and(t >= 0, t < SEG_PER_SC)
                idx2_v[pl.ds(k * L, L)] = jnp.where(valid, t, GARBAGE)
            pltpu.sync_copy(ones_v, cnts_sh.at[idx2_v], add=True)

    _loads(0, idx_v, sem_a)

    def _pair(j2, _):
        j = j2 * 2
        _loads(j + 1, idx_b, sem_b)
        _scatter(j, idx_v, sem_a)
        _loads(j + 2, idx_v, sem_a)
        _scatter(j + 1, idx_b, sem_b)
        return 0
    lax.fori_loop(0, NB // 2, _pair, 0)

    _scatter(NB - 1, idx_v, sem_a)
    plsc.subcore_barrier()

    g0 = c * SEG_PER_SC + s * CPW
    for t in range(CPW // ZB):
        pltpu.sync_copy(cnts_sh.at[pl.ds(s * CPW + t * ZB, ZB)],
                        zbuf)
        pltpu.sync_copy(zbuf, cnt_hbm.at[pl.ds(g0 + t * ZB, ZB)])


def _sums_body(x_hbm, idx_hbm, cnt_hbm, out_hbm, sums_sh, idx_v, idx_b,
               xbuf, xbuf_b, zbuf, rowbuf, cbuf, sem_a, sem_b):
    c = lax.axis_index("c")
    s = lax.axis_index("s")

    zero_v = jnp.zeros((L,), jnp.float32)

    def _fill(i, _):
        def _col(k, _):
            zbuf[i, pl.ds(k * L, L)] = zero_v
            return 0
        lax.fori_loop(0, DH // L, _col, 0)
        return 0
    lax.fori_loop(0, ZB, _fill, 0)

    z0 = s * ZPW2
    for t in range(ZPW2 // ZB):
        pltpu.sync_copy(zbuf, sums_sh.at[pl.ds(z0 + t * ZB, ZB)])
    plsc.subcore_barrier()

    e0 = s * EPW
    col = c * DH

    def _loads(j, ibuf, xb, sem):
        pltpu.async_copy(idx_hbm.at[pl.ds(e0 + j * B, B)], ibuf, sem)
        pltpu.async_copy(x_hbm.at[pl.ds(e0 + j * B, B), pl.ds(col, DH)],
                         xb, sem)

    def _waits(j, ibuf, xb, sem):
        pltpu.make_async_copy(idx_hbm.at[pl.ds(e0 + j * B, B)], ibuf,
                              sem).wait()
        pltpu.make_async_copy(x_hbm.at[pl.ds(e0 + j * B, B),
                                       pl.ds(col, DH)], xb, sem).wait()

    _loads(0, idx_v, xbuf, sem_a)

    def _pair(j2, _):
        j = j2 * 2
        _waits(j, idx_v, xbuf, sem_a)
        _loads(j + 1, idx_b, xbuf_b, sem_b)
        pltpu.sync_copy(xbuf, sums_sh.at[idx_v], add=True)
        _waits(j + 1, idx_b, xbuf_b, sem_b)
        _loads(j + 2, idx_v, xbuf, sem_a)
        pltpu.sync_copy(xbuf_b, sums_sh.at[idx_b], add=True)
        return 0
    lax.fori_loop(0, NB // 2, _pair, 0)

    _waits(NB - 1, idx_v, xbuf, sem_a)
    pltpu.sync_copy(xbuf, sums_sh.at[idx_v], add=True)
    plsc.subcore_barrier()

    nb_mine = jnp.where(s == 0, NDB - (NS - 1) * (NDB // NS), NDB // NS)

    def _div_batch(m, _):
        r0 = (s + m * NS) * DB
        pltpu.sync_copy(sums_sh.at[pl.ds(r0, DB)], rowbuf)
        pltpu.sync_copy(cnt_hbm.at[pl.ds(r0, DB)], cbuf)

        def _row(i, _):
            inv = 1.0 / jnp.maximum(cbuf[i, pl.ds(0, L)], 1.0)

            def _col(k, _):
                rowbuf[i, pl.ds(k * L, L)] = rowbuf[i, pl.ds(k * L, L)] * inv
                return 0
            lax.fori_loop(0, DH // L, _col, 0)
            return 0
        lax.fori_loop(0, DB, _row, 0)

        pltpu.sync_copy(rowbuf,
                        out_hbm.at[pl.ds(r0, DB), pl.ds(c * DH, DH)])
        return 0
    lax.fori_loop(0, nb_mine, _div_batch, 0)


def kernel(x, indices):
    idx = indices.astype(jnp.int32)
    mesh = plsc.VectorSubcoreMesh(
        core_axis_name="c", subcore_axis_name="s",
        num_cores=NC, num_subcores=NS)

    k1 = pl.kernel(
        _counts_body,
        out_type=jax.ShapeDtypeStruct((2 * SEG_PER_SC, DH), jnp.float32),
        mesh=mesh,
        scratch_types=[
            pltpu.VMEM_SHARED((CNT_ROWS, DH), jnp.float32),
            pltpu.VMEM((B,), jnp.int32),
            pltpu.VMEM((B,), jnp.int32),
            pltpu.VMEM((B,), jnp.int32),
            pltpu.VMEM((B, DH), jnp.float32),
            pltpu.VMEM((ZB, DH), jnp.float32),
            pltpu.SemaphoreType.DMA,
            pltpu.SemaphoreType.DMA,
        ],
    )
    counts = k1(idx)

    k2 = pl.kernel(
        _sums_body,
        out_type=jax.ShapeDtypeStruct((DIM_SIZE, D_FEAT), jnp.float32),
        mesh=mesh,
        scratch_types=[
            pltpu.VMEM_SHARED((NSEG_PAD, DH), jnp.float32),
            pltpu.VMEM((B,), jnp.int32),
            pltpu.VMEM((B,), jnp.int32),
            pltpu.VMEM((B, DH), jnp.float32),
            pltpu.VMEM((B, DH), jnp.float32),
            pltpu.VMEM((ZB, DH), jnp.float32),
            pltpu.VMEM((DB, DH), jnp.float32),
            pltpu.VMEM((DB, DH), jnp.float32),
            pltpu.SemaphoreType.DMA,
            pltpu.SemaphoreType.DMA,
        ],
    )
    return k2(x, idx, counts)


# speedup vs baseline: 3.9937x; 1.0532x over previous
import jax
import jax.numpy as jnp
from jax import lax
from jax.experimental import pallas as pl
from jax.experimental.pallas import tpu as pltpu
from jax.experimental.pallas import tpu_sc as plsc

N_EDGES = 160000
D_FEAT = 256
DIM_SIZE = 10000

NC = 2
NS = 16
L = 16
DH = D_FEAT // NC
EPW = N_EDGES // NS
B = 80
NB = EPW // B

SEG_PER_SC = 5120
GARBAGE = SEG_PER_SC
CNT_ROWS = SEG_PER_SC + 128
ZPW1 = CNT_ROWS // NS
CPW = SEG_PER_SC // NS

NSEG_PAD = 10240
ZPW2 = NSEG_PAD // NS
ZB = 64
DB = 16
NDB_PAD = NSEG_PAD // DB
DPW = NDB_PAD // NS


def _counts_body(idx_hbm, cnt_hbm, cnts_sh, idx_v, idx_b, idx2_v, ones_v,
                 zbuf, sem_a, sem_b):
    c = lax.axis_index("c")
    s = lax.axis_index("s")
    lo = c * SEG_PER_SC

    zero_v = jnp.zeros((L,), jnp.float32)
    one_v = jnp.ones((L,), jnp.float32)

    def _fill_ones(i, _):
        def _col(k, _):
            ones_v[i, pl.ds(k * L, L)] = one_v
            return 0
        lax.fori_loop(0, DH // L, _col, 0)
        return 0
    lax.fori_loop(0, B, _fill_ones, 0)

    def _fill(i, _):
        def _col(k, _):
            zbuf[i, pl.ds(k * L, L)] = zero_v
            return 0
        lax.fori_loop(0, DH // L, _col, 0)
        return 0
    lax.fori_loop(0, ZB, _fill, 0)

    z0 = s * ZPW1
    for t in range(ZPW1 // ZB):
        pltpu.sync_copy(zbuf, cnts_sh.at[pl.ds(z0 + t * ZB, ZB)])
    pltpu.sync_copy(zbuf.at[pl.ds(0, ZPW1 % ZB)],
                    cnts_sh.at[pl.ds(z0 + (ZPW1 // ZB) * ZB, ZPW1 % ZB)])
    plsc.subcore_barrier()

    e0 = s * EPW
    hi = lo + SEG_PER_SC

    def _loads(j, buf, sem):
        pltpu.async_copy(idx_hbm.at[pl.ds(e0 + j * B, B)], buf, sem)

    def _scatter(j, buf, sem):
        pltpu.make_async_copy(idx_hbm.at[pl.ds(e0 + j * B, B)], buf,
                              sem).wait()
        first = buf[pl.ds(0, L)][0]
        last = buf[pl.ds(B - L, L)][L - 1]

        @pl.when(jnp.logical_and(first < hi, last >= lo))
        def _():
            for k in range(B // L):
                v = buf[pl.ds(k * L, L)]
                t = v - lo
                valid = jnp.logical_and(t >= 0, t < SEG_PER_SC)
                idx2_v[pl.ds(k * L, L)] = jnp.where(valid, t, GARBAGE)
            pltpu.sync_copy(ones_v, cnts_sh.at[idx2_v], add=True)

    _loads(0, idx_v, sem_a)

    def _pair(j2, _):
        j = j2 * 2
        _loads(j + 1, idx_b, sem_b)
        _scatter(j, idx_v, sem_a)
        _loads(j + 2, idx_v, sem_a)
        _scatter(j + 1, idx_b, sem_b)
        return 0
    lax.fori_loop(0, NB // 2, _pair, 0)

    _scatter(NB - 1, idx_v, sem_a)
    plsc.subcore_barrier()

    g0 = c * SEG_PER_SC + s * CPW
    for t in range(CPW // ZB):
        pltpu.sync_copy(cnts_sh.at[pl.ds(s * CPW + t * ZB, ZB)],
                        zbuf)
        pltpu.sync_copy(zbuf, cnt_hbm.at[pl.ds(g0 + t * ZB, ZB)])


def _sums_body(x_hbm, idx_hbm, cnt_hbm, out_hbm, sums_sh, idx_v, idx_b,
               xbuf, xbuf_b, zbuf, rowbuf, rowbuf_b, cbuf, cbuf_b,
               sem_a, sem_b):
    c = lax.axis_index("c")
    s = lax.axis_index("s")

    zero_v = jnp.zeros((L,), jnp.float32)

    def _fill(i, _):
        def _col(k, _):
            zbuf[i, pl.ds(k * L, L)] = zero_v
            return 0
        lax.fori_loop(0, DH // L, _col, 0)
        return 0
    lax.fori_loop(0, ZB, _fill, 0)

    z0 = s * ZPW2
    for t in range(ZPW2 // ZB):
        pltpu.sync_copy(zbuf, sums_sh.at[pl.ds(z0 + t * ZB, ZB)])
    plsc.subcore_barrier()

    e0 = s * EPW
    col = c * DH

    def _loads(j, ibuf, xb, sem):
        pltpu.async_copy(idx_hbm.at[pl.ds(e0 + j * B, B)], ibuf, sem)
        pltpu.async_copy(x_hbm.at[pl.ds(e0 + j * B, B), pl.ds(col, DH)],
                         xb, sem)

    def _waits(j, ibuf, xb, sem):
        pltpu.make_async_copy(idx_hbm.at[pl.ds(e0 + j * B, B)], ibuf,
                              sem).wait()
        pltpu.make_async_copy(x_hbm.at[pl.ds(e0 + j * B, B),
                                       pl.ds(col, DH)], xb, sem).wait()

    _loads(0, idx_v, xbuf, sem_a)

    def _pair(j2, _):
        j = j2 * 2
        _waits(j, idx_v, xbuf, sem_a)
        _loads(j + 1, idx_b, xbuf_b, sem_b)
        pltpu.sync_copy(xbuf, sums_sh.at[idx_v], add=True)
        _waits(j + 1, idx_b, xbuf_b, sem_b)
        _loads(j + 2, idx_v, xbuf, sem_a)
        pltpu.sync_copy(xbuf_b, sums_sh.at[idx_b], add=True)
        return 0
    lax.fori_loop(0, NB // 2, _pair, 0)

    _waits(NB - 1, idx_v, xbuf, sem_a)
    pltpu.sync_copy(xbuf, sums_sh.at[idx_v], add=True)
    plsc.subcore_barrier()

    def _cnt_issue(m, cb, sem):
        r0 = (s + m * NS) * DB
        pltpu.async_copy(cnt_hbm.at[pl.ds(r0, DB)], cb, sem)

    def _div_finish(m, cb, sem):
        r0 = (s + m * NS) * DB
        pltpu.sync_copy(sums_sh.at[pl.ds(r0, DB)], rowbuf)
        pltpu.make_async_copy(cnt_hbm.at[pl.ds(r0, DB)], cb, sem).wait()

        def _row(i, _):
            inv = 1.0 / jnp.maximum(cb[i, pl.ds(0, L)], 1.0)

            def _col(k, _):
                rowbuf[i, pl.ds(k * L, L)] = rowbuf[i, pl.ds(k * L, L)] * inv
                return 0
            lax.fori_loop(0, DH // L, _col, 0)
            return 0
        lax.fori_loop(0, DB, _row, 0)

        pltpu.sync_copy(rowbuf,
                        out_hbm.at[pl.ds(r0, DB), pl.ds(c * DH, DH)])

    _cnt_issue(0, cbuf, sem_a)

    def _div_pair(m2, _):
        m = m2 * 2
        _cnt_issue(m + 1, cbuf_b, sem_b)
        _div_finish(m, cbuf, sem_a)
        _cnt_issue(m + 2, cbuf, sem_a)
        _div_finish(m + 1, cbuf_b, sem_b)
        return 0
    lax.fori_loop(0, DPW // 2 - 1, _div_pair, 0)

    _cnt_issue(DPW - 1, cbuf_b, sem_b)
    _div_finish(DPW - 2, cbuf, sem_a)
    _div_finish(DPW - 1, cbuf_b, sem_b)


def kernel(x, indices):
    idx = indices.astype(jnp.int32)
    mesh = plsc.VectorSubcoreMesh(
        core_axis_name="c", subcore_axis_name="s",
        num_cores=NC, num_subcores=NS)

    k1 = pl.kernel(
        _counts_body,
        out_type=jax.ShapeDtypeStruct((2 * SEG_PER_SC, DH), jnp.float32),
        mesh=mesh,
        scratch_types=[
            pltpu.VMEM_SHARED((CNT_ROWS, DH), jnp.float32),
            pltpu.VMEM((B,), jnp.int32),
            pltpu.VMEM((B,), jnp.int32),
            pltpu.VMEM((B,), jnp.int32),
            pltpu.VMEM((B, DH), jnp.float32),
            pltpu.VMEM((ZB, DH), jnp.float32),
            pltpu.SemaphoreType.DMA,
            pltpu.SemaphoreType.DMA,
        ],
    )
    counts = k1(idx)

    k2 = pl.kernel(
        _sums_body,
        out_type=jax.ShapeDtypeStruct((NSEG_PAD, D_FEAT), jnp.float32),
        mesh=mesh,
        scratch_types=[
            pltpu.VMEM_SHARED((NSEG_PAD, DH), jnp.float32),
            pltpu.VMEM((B,), jnp.int32),
            pltpu.VMEM((B,), jnp.int32),
            pltpu.VMEM((B, DH), jnp.float32),
            pltpu.VMEM((B, DH), jnp.float32),
            pltpu.VMEM((ZB, DH), jnp.float32),
            pltpu.VMEM((DB, DH), jnp.float32),
            pltpu.VMEM((DB, DH), jnp.float32),
            pltpu.VMEM((DB, DH), jnp.float32),
            pltpu.VMEM((DB, DH), jnp.float32),
            pltpu.SemaphoreType.DMA,
            pltpu.SemaphoreType.DMA,
        ],
    )
    return k2(x, idx, counts)[:DIM_SIZE]
